# baseline (speedup 1.0000x reference)
import functools

import numpy as np
import jax
import jax.numpy as jnp
from jax import lax
from jax.experimental import pallas as pl
from jax.experimental.pallas import tpu as pltpu
from jax.experimental.pallas import tpu_sc as plsc

NUM_ANCHORS = 16
K = 10
FEATURE_DIM = 128
ROW_TILE = 1280

NUM_WORKERS = 32
SC_CHUNK = 500


def _head_kernel(x_ref, wb_ref, bb_ref, cw_ref, cb_ref,
                 bbox_ref, code_ref, vals_ref, mask_ref):
    m_rows = x_ref.shape[0]
    x = x_ref[:]
    bbox = jnp.dot(x, wb_ref[:], preferred_element_type=jnp.float32) + bb_ref[:]
    logits_t = lax.dot_general(cw_ref[:], x, (((0,), (1,)), ((), ())),
                               preferred_element_type=jnp.float32) + cb_ref[:]
    c = jax.nn.sigmoid(logits_t)

    iota_a = lax.broadcasted_iota(jnp.int32, (NUM_ANCHORS, m_rows), 0)
    vals_rows, code_rows = [], []
    for _ in range(K):
        m = jnp.max(c, axis=0, keepdims=True)
        idxk = jnp.min(jnp.where(c == m, iota_a, NUM_ANCHORS),
                       axis=0, keepdims=True)
        vals_rows.append(m)
        code_rows.append(jnp.where(m > 0.5, (4 * idxk).astype(jnp.float32), -1.0))
        c = jnp.where(iota_a == idxk, -1.0, c)

    vpad = jnp.zeros((NUM_ANCHORS - K, m_rows), jnp.float32)
    cpad = jnp.full((NUM_ANCHORS - K, m_rows), -1.0, jnp.float32)
    vals16 = jnp.concatenate(vals_rows + [vpad], axis=0).T
    code16 = jnp.concatenate(code_rows + [cpad], axis=0).T

    vals = vals16[:, :K]
    bbox_ref[:] = bbox
    code_ref[:] = code16.astype(jnp.int32)
    vals_ref[:] = vals
    mask_ref[:] = vals > 0.5


def _tc_stage(x, bbox_W, bbox_b, conf_W, conf_b):
    R = x.shape[0]
    bb = bbox_b[None, :]
    cbT = conf_b[:, None]
    grid = (R // ROW_TILE,)
    return pl.pallas_call(
        _head_kernel,
        grid=grid,
        in_specs=[
            pl.BlockSpec((ROW_TILE, FEATURE_DIM), lambda i: (i, 0)),
            pl.BlockSpec((FEATURE_DIM, NUM_ANCHORS * 4), lambda i: (0, 0)),
            pl.BlockSpec((1, NUM_ANCHORS * 4), lambda i: (0, 0)),
            pl.BlockSpec((FEATURE_DIM, NUM_ANCHORS), lambda i: (0, 0)),
            pl.BlockSpec((NUM_ANCHORS, 1), lambda i: (0, 0)),
        ],
        out_specs=[
            pl.BlockSpec((ROW_TILE, NUM_ANCHORS * 4), lambda i: (i, 0)),
            pl.BlockSpec((ROW_TILE, NUM_ANCHORS), lambda i: (i, 0)),
            pl.BlockSpec((ROW_TILE, K), lambda i: (i, 0)),
            pl.BlockSpec((ROW_TILE, K), lambda i: (i, 0)),
        ],
        out_shape=[
            jax.ShapeDtypeStruct((R, NUM_ANCHORS * 4), jnp.float32),
            jax.ShapeDtypeStruct((R, NUM_ANCHORS), jnp.int32),
            jax.ShapeDtypeStruct((R, K), jnp.float32),
            jax.ShapeDtypeStruct((R, K), jnp.bool_),
        ],
        compiler_params=pltpu.CompilerParams(
            dimension_semantics=("parallel",),
        ),
    )(x, bbox_W, bb, conf_W, cbT)


def _sc_stage(code_flat, bbox_flat, R):
    rows_per_w = R // NUM_WORKERS
    n_chunks = rows_per_w // SC_CHUNK
    C = SC_CHUNK
    mesh = plsc.VectorSubcoreMesh(core_axis_name="c", subcore_axis_name="s")

    @functools.partial(
        pl.kernel,
        out_type=jax.ShapeDtypeStruct((R * K * 4,), jnp.float32),
        mesh=mesh,
        scratch_types=[
            pltpu.VMEM((C * 16,), jnp.int32),
            pltpu.VMEM((C * 64 + 16,), jnp.float32),
            pltpu.VMEM((C * 40 + 16,), jnp.float32),
        ],
        compiler_params=pltpu.CompilerParams(needs_layout_passes=False),
    )
    def sc_gather(code_hbm, bbox_hbm, det_hbm, code_v, bbox_v, det_v):
        wid = lax.axis_index("s") * 2 + lax.axis_index("c")
        row0 = wid * rows_per_w
        lane = lax.iota(jnp.int32, 16)
        cmod = lane & 3
        kbase = lane >> 2
        zeros16 = jnp.zeros((16,), jnp.float32)
        bbox_v[pl.ds(C * 64, 16)] = zeros16

        def chunk_body(ci, _):
            r0 = row0 + ci * C
            pltpu.sync_copy(code_hbm.at[pl.ds(r0 * 16, C * 16)], code_v)
            pltpu.sync_copy(bbox_hbm.at[pl.ds(r0 * 64, C * 64)],
                            bbox_v.at[pl.ds(0, C * 64)])

            def row_body(r, _):
                b64 = r * 64
                for j in range(3):
                    cidx = kbase + (4 * j + r * 16)
                    codes = plsc.load_gather(code_v, [cidx])
                    src = jnp.where(codes >= 0, codes + b64, C * 64) + cmod
                    det_v[pl.ds(r * 40 + 16 * j, 16)] = plsc.load_gather(bbox_v, [src])
                return 0

            lax.fori_loop(0, C, row_body, 0)
            pltpu.sync_copy(det_v.at[pl.ds(0, C * 40)],
                            det_hbm.at[pl.ds(r0 * 40, C * 40)])
            return 0

        lax.fori_loop(0, n_chunks, chunk_body, 0)

    return sc_gather(code_flat, bbox_flat)


@functools.partial(jax.jit, static_argnames=())
def kernel(features, bbox_W, bbox_b, conf_W, conf_b):
    B, T, F = features.shape
    R = B * T
    x = features.reshape(R, F)
    bbox, code, vals, mask = _tc_stage(x, bbox_W, bbox_b, conf_W, conf_b)
    det_flat = _sc_stage(code.reshape(-1), bbox.reshape(-1), R)
    return (det_flat.reshape(B, T, K, 4), vals.reshape(B, T, K),
            mask.reshape(B, T, K))
